# baseline (speedup 1.0000x reference)
import functools
import jax
import jax.numpy as jnp
from jax import lax
from jax.experimental import pallas as pl
from jax.experimental.pallas import tpu as pltpu

N_NODES = 100000
E_EDGES = 1600000


def _matmul_bias_kernel(x_ref, w_ref, b_ref, o_ref):
    o_ref[...] = jnp.dot(x_ref[...], w_ref[...],
                         preferred_element_type=jnp.float32) + b_ref[...]


def _matmul_bias(x, w, b, block_rows):
    n, di = x.shape
    do = w.shape[1]
    grid = (n // block_rows,)
    return pl.pallas_call(
        _matmul_bias_kernel,
        grid=grid,
        in_specs=[
            pl.BlockSpec((block_rows, di), lambda i: (i, 0)),
            pl.BlockSpec((di, do), lambda i: (0, 0)),
            pl.BlockSpec((1, do), lambda i: (0, 0)),
        ],
        out_specs=pl.BlockSpec((block_rows, do), lambda i: (i, 0)),
        out_shape=jax.ShapeDtypeStruct((n, do), jnp.float32),
    )(x, w, b.reshape(1, do))


def _combine_kernel(x_ref, w_ref, b_ref, s0_ref, s1_ref, c0_ref, c1_ref,
                    o_ref, *, relu):
    out = jnp.dot(x_ref[...], w_ref[...],
                  preferred_element_type=jnp.float32) + b_ref[...]
    out = out + s0_ref[...] / jnp.maximum(c0_ref[...], 1.0)
    out = out + s1_ref[...] / jnp.maximum(c1_ref[...], 1.0)
    if relu:
        out = jnp.maximum(out, 0.0)
    o_ref[...] = out


def _combine(x, root, bias, s0, s1, c0, c1, relu, block_rows=4000):
    n, di = x.shape
    do = root.shape[1]
    grid = (n // block_rows,)
    return pl.pallas_call(
        functools.partial(_combine_kernel, relu=relu),
        grid=grid,
        in_specs=[
            pl.BlockSpec((block_rows, di), lambda i: (i, 0)),
            pl.BlockSpec((di, do), lambda i: (0, 0)),
            pl.BlockSpec((1, do), lambda i: (0, 0)),
            pl.BlockSpec((block_rows, do), lambda i: (i, 0)),
            pl.BlockSpec((block_rows, do), lambda i: (i, 0)),
            pl.BlockSpec((block_rows, 1), lambda i: (i, 0)),
            pl.BlockSpec((block_rows, 1), lambda i: (i, 0)),
        ],
        out_specs=pl.BlockSpec((block_rows, do), lambda i: (i, 0)),
        out_shape=jax.ShapeDtypeStruct((n, do), jnp.float32),
    )(x, root, bias.reshape(1, do), s0, s1, c0, c1)


def _decode_kernel(zs_ref, zd_ref, et_ref, w0_ref, b0_ref, w1_ref, b1_ref,
                   o_ref):
    zs = zs_ref[...]
    t0 = jnp.dot(zs, w0_ref[...], preferred_element_type=jnp.float32) + b0_ref[...]
    t1 = jnp.dot(zs, w1_ref[...], preferred_element_type=jnp.float32) + b1_ref[...]
    sel = jnp.where(et_ref[...] == 0, t0, t1)
    o_ref[...] = jnp.sum(sel * zd_ref[...], axis=-1, keepdims=True)


def _decode_dots(zs, zd, et, w0, b0, w1, b1, block_rows=8000):
    e, de = zs.shape
    grid = (e // block_rows,)
    return pl.pallas_call(
        _decode_kernel,
        grid=grid,
        in_specs=[
            pl.BlockSpec((block_rows, de), lambda i: (i, 0)),
            pl.BlockSpec((block_rows, de), lambda i: (i, 0)),
            pl.BlockSpec((block_rows, 1), lambda i: (i, 0)),
            pl.BlockSpec((de, de), lambda i: (0, 0)),
            pl.BlockSpec((1, de), lambda i: (0, 0)),
            pl.BlockSpec((de, de), lambda i: (0, 0)),
            pl.BlockSpec((1, de), lambda i: (0, 0)),
        ],
        out_specs=pl.BlockSpec((block_rows, 1), lambda i: (i, 0)),
        out_shape=jax.ShapeDtypeStruct((e, 1), jnp.float32),
    )(zs, zd, et.reshape(e, 1), w0, b0.reshape(1, de), w1, b1.reshape(1, de))


def _rgcn_layer(x, srcrow, dstrow, c0, c1, basis, comp, root, bias, relu):
    w = jnp.einsum('rb,bio->rio', comp, basis)
    h = jnp.concatenate([x @ w[0], x @ w[1]], axis=0)
    msg = h[srcrow]
    s = jnp.zeros((2 * N_NODES, h.shape[1]), jnp.float32).at[dstrow].add(msg)
    return _combine(x, root, bias, s[:N_NODES], s[N_NODES:], c0, c1, relu)


def kernel(x_paper, x_mesh, tp_W, tp_b, tm_W, tm_b, basis1, comp1, root1, bias1, basis2, comp2, root2, bias2, basis3, comp3, root3, bias3, basis4, comp4, root4, bias4, dpp_W, dpp_b, dpm_W, dpm_b, train_pos_edge_index, train_pos_edge_type, pos_edge_index, pos_edge_type, neg_edge_index, neg_edge_type):
    n = N_NODES
    xp = _matmul_bias(x_paper, tp_W, tp_b, block_rows=2000)
    xm = _matmul_bias(x_mesh, tm_W, tm_b, block_rows=2000)
    x = jnp.concatenate([xp, xm], axis=0)

    et = train_pos_edge_type
    src = train_pos_edge_index[0]
    dst = train_pos_edge_index[1]
    srcrow = et * n + src
    dstrow = et * n + dst
    c = jnp.zeros((2 * n,), jnp.float32).at[dstrow].add(1.0)
    c0 = c[:n].reshape(n, 1)
    c1 = c[n:].reshape(n, 1)

    x = _rgcn_layer(x, srcrow, dstrow, c0, c1, basis1, comp1, root1, bias1, True)
    x = _rgcn_layer(x, srcrow, dstrow, c0, c1, basis2, comp2, root2, bias2, True)
    x = _rgcn_layer(x, srcrow, dstrow, c0, c1, basis3, comp3, root3, bias3, True)
    z = _rgcn_layer(x, srcrow, dstrow, c0, c1, basis4, comp4, root4, bias4, False)

    def half(ei, et_h):
        s = _decode_dots(z[ei[0]], z[ei[1]], et_h, dpp_W, dpp_b, dpm_W, dpm_b)
        s = s.reshape(-1)
        is0 = (et_h == 0)
        n0 = jnp.sum(is0.astype(jnp.int32))
        r0 = jnp.cumsum(is0.astype(jnp.int32)) - 1
        r1 = jnp.cumsum(1 - is0.astype(jnp.int32)) - 1
        pos = jnp.where(is0, r0, n0 + r1)
        return jnp.zeros_like(s).at[pos].set(s)

    lp = half(pos_edge_index, pos_edge_type)
    ln = half(neg_edge_index, neg_edge_type)
    return jnp.concatenate([lp, ln], axis=0)
